# baseline (speedup 1.0000x reference)
import jax
import jax.numpy as jnp
from jax.experimental import pallas as pl
from jax.experimental.pallas import tpu as pltpu

_V = 8192


def _ce_kernel(x_ref, y_ref, row_ref, logits_ref, loss_ref, acc_ref):
    i = pl.program_id(0)
    n = pl.num_programs(0)
    row = row_ref[...]
    logits_ref[...] = row
    m = jnp.max(row)
    s = jnp.sum(jnp.exp(row - m))
    lse = m + jnp.log(s)
    yi = y_ref[i]
    col = jax.lax.broadcasted_iota(jnp.int32, row.shape, 2)
    val = jnp.sum(jnp.where(col == yi, row, 0.0))
    nll = lse - val

    @pl.when(i == 0)
    def _init():
        acc_ref[0] = 0.0

    acc_ref[0] += nll

    @pl.when(i == n - 1)
    def _fin():
        loss_ref[...] = jnp.full((1, 1), acc_ref[0] / n, jnp.float32)


def kernel(x, y, table):
    xf = x.reshape(-1).astype(jnp.int32)
    yf = y.reshape(-1).astype(jnp.int32)
    ntok = xf.shape[0]
    table3 = table.reshape(_V, 1, _V)

    grid_spec = pltpu.PrefetchScalarGridSpec(
        num_scalar_prefetch=2,
        grid=(ntok,),
        in_specs=[
            pl.BlockSpec((1, 1, _V), lambda i, xr, yr: (xr[i], 0, 0)),
        ],
        out_specs=[
            pl.BlockSpec((1, 1, _V), lambda i, xr, yr: (i, 0, 0)),
            pl.BlockSpec((1, 1), lambda i, xr, yr: (0, 0)),
        ],
        scratch_shapes=[pltpu.SMEM((1,), jnp.float32)],
    )

    logits, loss = pl.pallas_call(
        _ce_kernel,
        grid_spec=grid_spec,
        out_shape=[
            jax.ShapeDtypeStruct((ntok, 1, _V), jnp.float32),
            jax.ShapeDtypeStruct((1, 1), jnp.float32),
        ],
    )(xf, yf, table3)
    return (logits.reshape(ntok, _V), loss[0, 0])


# speedup vs baseline: 5.0075x; 5.0075x over previous
import jax
import jax.numpy as jnp
from jax.experimental import pallas as pl
from jax.experimental.pallas import tpu as pltpu

_V = 8192
_R = 8


def _ce_kernel(x_ref, y_ref, *refs):
    row_refs = refs[:_R]
    logits_ref, loss_ref = refs[_R], refs[_R + 1]
    acc_ref = refs[_R + 2]
    i = pl.program_id(0)
    n = pl.num_programs(0)

    rows = jnp.concatenate([r[0] for r in row_refs], axis=0)
    logits_ref[...] = rows

    m = jnp.max(rows, axis=1, keepdims=True)
    s = jnp.sum(jnp.exp(rows - m), axis=1, keepdims=True)
    lse = m + jnp.log(s)

    yv = jnp.concatenate(
        [jnp.full((1, 1), y_ref[_R * i + k], jnp.int32) for k in range(_R)],
        axis=0)
    col = jax.lax.broadcasted_iota(jnp.int32, rows.shape, 1)
    val = jnp.sum(jnp.where(col == yv, rows, 0.0), axis=1, keepdims=True)

    @pl.when(i == 0)
    def _init():
        acc_ref[0] = 0.0

    acc_ref[0] += jnp.sum(lse - val)

    @pl.when(i == n - 1)
    def _fin():
        loss_ref[...] = jnp.full((1, 1), acc_ref[0] / (n * _R), jnp.float32)


def kernel(x, y, table):
    xf = x.reshape(-1).astype(jnp.int32)
    yf = y.reshape(-1).astype(jnp.int32)
    ntok = xf.shape[0]
    table3 = table.reshape(_V, 1, _V)

    def _row_spec(k):
        return pl.BlockSpec((1, 1, _V), lambda i, xr, yr, _k=k: (xr[_R * i + _k], 0, 0))

    grid_spec = pltpu.PrefetchScalarGridSpec(
        num_scalar_prefetch=2,
        grid=(ntok // _R,),
        in_specs=[_row_spec(k) for k in range(_R)],
        out_specs=[
            pl.BlockSpec((_R, _V), lambda i, xr, yr: (i, 0)),
            pl.BlockSpec((1, 1), lambda i, xr, yr: (0, 0)),
        ],
        scratch_shapes=[pltpu.SMEM((1,), jnp.float32)],
    )

    logits, loss = pl.pallas_call(
        _ce_kernel,
        grid_spec=grid_spec,
        out_shape=[
            jax.ShapeDtypeStruct((ntok, _V), jnp.float32),
            jax.ShapeDtypeStruct((1, 1), jnp.float32),
        ],
    )(xf, yf, *([table3] * _R))
    return (logits, loss[0, 0])


# speedup vs baseline: 6.8932x; 1.3766x over previous
import jax
import jax.numpy as jnp
from jax.experimental import pallas as pl
from jax.experimental.pallas import tpu as pltpu

_V = 8192
_R = 16


def _ce_kernel(x_ref, y_ref, *refs):
    row_refs = refs[:_R]
    logits_ref, loss_ref = refs[_R], refs[_R + 1]
    acc_ref = refs[_R + 2]
    i = pl.program_id(0)
    n = pl.num_programs(0)

    rows = jnp.concatenate([r[0] for r in row_refs], axis=0)
    logits_ref[...] = rows

    m = jnp.max(rows, axis=1, keepdims=True)
    s = jnp.sum(jnp.exp(rows - m), axis=1, keepdims=True)
    lse = m + jnp.log(s)

    yv = jnp.concatenate(
        [jnp.full((1, 1), y_ref[_R * i + k], jnp.int32) for k in range(_R)],
        axis=0)
    col = jax.lax.broadcasted_iota(jnp.int32, rows.shape, 1)
    val = jnp.sum(jnp.where(col == yv, rows, 0.0), axis=1, keepdims=True)

    @pl.when(i == 0)
    def _init():
        acc_ref[0] = 0.0

    acc_ref[0] += jnp.sum(lse - val)

    @pl.when(i == n - 1)
    def _fin():
        loss_ref[...] = jnp.full((1, 1), acc_ref[0] / (n * _R), jnp.float32)


def kernel(x, y, table):
    xf = x.reshape(-1).astype(jnp.int32)
    yf = y.reshape(-1).astype(jnp.int32)
    ntok = xf.shape[0]
    table3 = table.reshape(_V, 1, _V)

    def _row_spec(k):
        return pl.BlockSpec((1, 1, _V), lambda i, xr, yr, _k=k: (xr[_R * i + _k], 0, 0))

    grid_spec = pltpu.PrefetchScalarGridSpec(
        num_scalar_prefetch=2,
        grid=(ntok // _R,),
        in_specs=[_row_spec(k) for k in range(_R)],
        out_specs=[
            pl.BlockSpec((_R, _V), lambda i, xr, yr: (i, 0)),
            pl.BlockSpec((1, 1), lambda i, xr, yr: (0, 0)),
        ],
        scratch_shapes=[pltpu.SMEM((1,), jnp.float32)],
    )

    logits, loss = pl.pallas_call(
        _ce_kernel,
        grid_spec=grid_spec,
        out_shape=[
            jax.ShapeDtypeStruct((ntok, _V), jnp.float32),
            jax.ShapeDtypeStruct((1, 1), jnp.float32),
        ],
    )(xf, yf, *([table3] * _R))
    return (logits, loss[0, 0])


# speedup vs baseline: 7.9404x; 1.1519x over previous
import jax
import jax.numpy as jnp
from jax.experimental import pallas as pl
from jax.experimental.pallas import tpu as pltpu

_V = 8192
_R = 32


def _ce_kernel(x_ref, y_ref, *refs):
    row_refs = refs[:_R]
    logits_ref, loss_ref = refs[_R], refs[_R + 1]
    acc_ref = refs[_R + 2]
    i = pl.program_id(0)
    n = pl.num_programs(0)

    rows = jnp.concatenate([r[...] for r in row_refs], axis=0)
    logits_ref[...] = rows

    m = jnp.max(rows, axis=(1, 2), keepdims=True)
    s = jnp.sum(jnp.exp(rows - m), axis=(1, 2), keepdims=True)
    lse = m + jnp.log(s)

    yv = jnp.concatenate(
        [jnp.full((1, 1, 1), y_ref[_R * i + k], jnp.int32) for k in range(_R)],
        axis=0)
    idx = (jax.lax.broadcasted_iota(jnp.int32, rows.shape, 1) * 128
           + jax.lax.broadcasted_iota(jnp.int32, rows.shape, 2))
    val = jnp.sum(jnp.where(idx == yv, rows, 0.0), axis=(1, 2), keepdims=True)

    @pl.when(i == 0)
    def _init():
        acc_ref[0] = 0.0

    acc_ref[0] += jnp.sum(lse - val)

    @pl.when(i == n - 1)
    def _fin():
        loss_ref[...] = jnp.full((1, 1), acc_ref[0] / (n * _R), jnp.float32)


def kernel(x, y, table):
    xf = x.reshape(-1).astype(jnp.int32)
    yf = y.reshape(-1).astype(jnp.int32)
    ntok = xf.shape[0]
    table4 = table.reshape(_V, 64, 128)

    def _row_spec(k):
        return pl.BlockSpec((1, 64, 128),
                            lambda i, xr, yr, _k=k: (xr[_R * i + _k], 0, 0))

    grid_spec = pltpu.PrefetchScalarGridSpec(
        num_scalar_prefetch=2,
        grid=(ntok // _R,),
        in_specs=[_row_spec(k) for k in range(_R)],
        out_specs=[
            pl.BlockSpec((_R, 64, 128), lambda i, xr, yr: (i, 0, 0)),
            pl.BlockSpec((1, 1), lambda i, xr, yr: (0, 0)),
        ],
        scratch_shapes=[pltpu.SMEM((1,), jnp.float32)],
    )

    logits, loss = pl.pallas_call(
        _ce_kernel,
        grid_spec=grid_spec,
        out_shape=[
            jax.ShapeDtypeStruct((ntok, 64, 128), jnp.float32),
            jax.ShapeDtypeStruct((1, 1), jnp.float32),
        ],
    )(xf, yf, *([table4] * _R))
    return (logits.reshape(ntok, _V), loss[0, 0])


# speedup vs baseline: 8.5143x; 1.0723x over previous
import jax
import jax.numpy as jnp
from jax.experimental import pallas as pl
from jax.experimental.pallas import tpu as pltpu

_V = 8192
_R = 64


def _ce_kernel(x_ref, y_ref, *refs):
    row_refs = refs[:_R]
    logits_ref, loss_ref = refs[_R], refs[_R + 1]
    acc_ref = refs[_R + 2]
    i = pl.program_id(0)
    n = pl.num_programs(0)

    rows = jnp.concatenate([r[...] for r in row_refs], axis=0)
    logits_ref[...] = rows

    m = jnp.max(rows, axis=(1, 2), keepdims=True)
    s = jnp.sum(jnp.exp(rows - m), axis=(1, 2), keepdims=True)
    lse = m + jnp.log(s)

    yv = jnp.concatenate(
        [jnp.full((1, 1, 1), y_ref[_R * i + k], jnp.int32) for k in range(_R)],
        axis=0)
    idx = (jax.lax.broadcasted_iota(jnp.int32, rows.shape, 1) * 128
           + jax.lax.broadcasted_iota(jnp.int32, rows.shape, 2))
    val = jnp.sum(jnp.where(idx == yv, rows, 0.0), axis=(1, 2), keepdims=True)

    @pl.when(i == 0)
    def _init():
        acc_ref[0] = 0.0

    acc_ref[0] += jnp.sum(lse - val)

    @pl.when(i == n - 1)
    def _fin():
        loss_ref[...] = jnp.full((1, 1), acc_ref[0] / (n * _R), jnp.float32)


def kernel(x, y, table):
    xf = x.reshape(-1).astype(jnp.int32)
    yf = y.reshape(-1).astype(jnp.int32)
    ntok = xf.shape[0]
    table4 = table.reshape(_V, 64, 128)

    def _row_spec(k):
        return pl.BlockSpec((1, 64, 128),
                            lambda i, xr, yr, _k=k: (xr[_R * i + _k], 0, 0))

    grid_spec = pltpu.PrefetchScalarGridSpec(
        num_scalar_prefetch=2,
        grid=(ntok // _R,),
        in_specs=[_row_spec(k) for k in range(_R)],
        out_specs=[
            pl.BlockSpec((_R, 64, 128), lambda i, xr, yr: (i, 0, 0)),
            pl.BlockSpec((1, 1), lambda i, xr, yr: (0, 0)),
        ],
        scratch_shapes=[pltpu.SMEM((1,), jnp.float32)],
    )

    logits, loss = pl.pallas_call(
        _ce_kernel,
        grid_spec=grid_spec,
        out_shape=[
            jax.ShapeDtypeStruct((ntok, 64, 128), jnp.float32),
            jax.ShapeDtypeStruct((1, 1), jnp.float32),
        ],
    )(xf, yf, *([table4] * _R))
    return (logits.reshape(ntok, _V), loss[0, 0])


# speedup vs baseline: 11.8596x; 1.3929x over previous
import functools

import jax
import jax.numpy as jnp
from jax import lax
from jax.experimental import pallas as pl
from jax.experimental.pallas import tpu as pltpu
from jax.experimental.pallas import tpu_sc as plsc

_V = 8192
_NC, _NS = 2, 16
_NW = _NC * _NS
_TPW = 256
_CH = 4
_NCHUNK = _TPW // _CH


def _gather_body(x2_hbm, table_hbm, out_hbm, idx_v, buf0, buf1,
                 si0, si1, so0, so1):
    wid = lax.axis_index("s") * _NC + lax.axis_index("c")
    base = wid * _TPW
    pltpu.sync_copy(x2_hbm.at[wid], idx_v)

    pltpu.async_copy(table_hbm.at[idx_v.at[0]], buf0, si0)
    pltpu.async_copy(table_hbm.at[idx_v.at[1]], buf1, si1)

    def body(o, carry):
        for b, (buf, si, so) in enumerate(((buf0, si0, so0),
                                           (buf1, si1, so1))):
            g = o * 2 + b
            dst = out_hbm.at[pl.ds(base + g * _CH, _CH)]
            pltpu.make_async_copy(table_hbm.at[idx_v.at[g]], buf, si).wait()
            pltpu.async_copy(buf, dst, so)
            pltpu.make_async_copy(buf, dst, so).wait()

            @pl.when(g + 2 < _NCHUNK)
            def _next():
                pltpu.async_copy(table_hbm.at[idx_v.at[g + 2]], buf, si)
        return carry

    lax.fori_loop(0, _NCHUNK // 2, body, 0)


def _lse_kernel(t_ref, lse_ref):
    t = t_ref[...]
    m = jnp.max(t, axis=1, keepdims=True)
    s = jnp.sum(jnp.exp(t - m), axis=1, keepdims=True)
    lse_ref[...] = m + jnp.log(s)


def _combine_body(x3_hbm, y3_hbm, table1_hbm, lse_hbm, out_hbm,
                  xv, yv, fv, tv, lv, accv, sem):
    wid = lax.axis_index("s") * _NC + lax.axis_index("c")
    pltpu.sync_copy(x3_hbm.at[wid], xv)
    pltpu.sync_copy(y3_hbm.at[wid], yv)
    for r in range(2):
        for j in range(8):
            c = j * 16
            fv[r, pl.ds(c, 16)] = xv[r, pl.ds(c, 16)] * _V + yv[r, pl.ds(c, 16)]
    for r in range(2):
        pltpu.async_copy(table1_hbm.at[fv.at[r]], tv.at[r], sem)
        pltpu.make_async_copy(table1_hbm.at[fv.at[r]], tv.at[r], sem).wait()
        pltpu.async_copy(lse_hbm.at[xv.at[r]], lv.at[r], sem)
        pltpu.make_async_copy(lse_hbm.at[xv.at[r]], lv.at[r], sem).wait()
    acc = jnp.zeros((16,), jnp.float32)
    for r in range(2):
        for j in range(8):
            c = j * 16
            acc = acc + (lv[r, pl.ds(c, 16)] - tv[r, pl.ds(c, 16)])
    accv[...] = acc
    pltpu.sync_copy(accv, out_hbm.at[wid])


def _loss_kernel(p_ref, loss_ref):
    loss_ref[...] = jnp.full((1, 1), jnp.sum(p_ref[...]) / (_NW * _TPW),
                             jnp.float32)


def kernel(x, y, table):
    xf = x.reshape(-1).astype(jnp.int32)
    yf = y.reshape(-1).astype(jnp.int32)
    ntok = xf.shape[0]
    mesh = plsc.VectorSubcoreMesh(core_axis_name="c", subcore_axis_name="s")

    x2 = xf.reshape(_NW, _NCHUNK, _CH)
    gather = pl.kernel(
        _gather_body,
        out_type=jax.ShapeDtypeStruct((ntok, _V), jnp.float32),
        mesh=mesh,
        scratch_types=[
            pltpu.VMEM((_NCHUNK, _CH), jnp.int32),
            pltpu.VMEM((_CH, _V), jnp.float32),
            pltpu.VMEM((_CH, _V), jnp.float32),
            pltpu.SemaphoreType.DMA,
            pltpu.SemaphoreType.DMA,
            pltpu.SemaphoreType.DMA,
            pltpu.SemaphoreType.DMA,
        ],
    )
    logits = gather(x2, table)

    _RB = 256
    lse = pl.pallas_call(
        _lse_kernel,
        grid=(_V // _RB,),
        in_specs=[pl.BlockSpec((_RB, _V), lambda i: (i, 0))],
        out_specs=pl.BlockSpec((_RB, 1), lambda i: (i, 0)),
        out_shape=jax.ShapeDtypeStruct((_V, 1), jnp.float32),
    )(table)

    x3 = xf.reshape(_NW, 2, 128)
    y3 = yf.reshape(_NW, 2, 128)
    combine = pl.kernel(
        _combine_body,
        out_type=jax.ShapeDtypeStruct((_NW, 16), jnp.float32),
        mesh=mesh,
        scratch_types=[
            pltpu.VMEM((2, 128), jnp.int32),
            pltpu.VMEM((2, 128), jnp.int32),
            pltpu.VMEM((2, 128), jnp.int32),
            pltpu.VMEM((2, 128), jnp.float32),
            pltpu.VMEM((2, 128), jnp.float32),
            pltpu.VMEM((16,), jnp.float32),
            pltpu.SemaphoreType.DMA,
        ],
    )
    partials = combine(x3, y3, table.reshape(_V * _V), lse.reshape(_V))

    loss = pl.pallas_call(
        _loss_kernel,
        out_shape=jax.ShapeDtypeStruct((1, 1), jnp.float32),
    )(partials)

    return (logits, loss[0, 0])


# speedup vs baseline: 18.7177x; 1.5783x over previous
import jax
import jax.numpy as jnp
from jax import lax
from jax.experimental import pallas as pl
from jax.experimental.pallas import tpu as pltpu
from jax.experimental.pallas import tpu_sc as plsc

_V = 8192
_NC, _NS = 2, 16
_NW = _NC * _NS
_TPW = 256
_CH = 4
_NCHUNK = _TPW // _CH


def _gather_body(x2_hbm, ycol_hbm, table_hbm, out_hbm, val_out_hbm,
                 idx_v, ycol_v, buf0, buf1, accv, si0, si1, so0, so1):
    wid = lax.axis_index("s") * _NC + lax.axis_index("c")
    base = wid * _TPW
    pltpu.sync_copy(x2_hbm.at[wid], idx_v)
    pltpu.sync_copy(ycol_hbm.at[wid], ycol_v)
    lane = lax.iota(jnp.int32, 16)

    pltpu.async_copy(table_hbm.at[idx_v.at[0]], buf0, si0)
    pltpu.async_copy(table_hbm.at[idx_v.at[1]], buf1, si1)

    def body(o, acc):
        for b, (buf, si, so) in enumerate(((buf0, si0, so0),
                                           (buf1, si1, so1))):
            g = o * 2 + b
            dst = out_hbm.at[pl.ds(base + g * _CH, _CH)]
            pltpu.make_async_copy(table_hbm.at[idx_v.at[g]], buf, si).wait()
            pltpu.async_copy(buf, dst, so)
            yrow = ycol_v[g]
            for r in range(_CH):
                yi = yrow[r]
                st = pl.multiple_of(yi & ~15, 16)
                w = buf[r, pl.ds(st, 16)]
                acc = acc + jnp.where(lane == (yi & 15), w, 0.0)
            pltpu.make_async_copy(buf, dst, so).wait()

            @pl.when(g + 2 < _NCHUNK)
            def _next():
                pltpu.async_copy(table_hbm.at[idx_v.at[g + 2]], buf, si)
        return acc

    acc = lax.fori_loop(0, _NCHUNK // 2, body, jnp.zeros((16,), jnp.float32))
    accv[...] = acc
    pltpu.sync_copy(accv, val_out_hbm.at[wid])


def _lse_kernel(t_ref, lse_ref):
    t = t_ref[...]
    m = jnp.max(t, axis=1, keepdims=True)
    s = jnp.sum(jnp.exp(t - m), axis=1, keepdims=True)
    lse_ref[...] = m + jnp.log(s)


def _lse_gather_body(x3_hbm, lse_hbm, out_hbm, xv, lv, accv, sem):
    wid = lax.axis_index("s") * _NC + lax.axis_index("c")
    pltpu.sync_copy(x3_hbm.at[wid], xv)
    for r in range(2):
        pltpu.async_copy(lse_hbm.at[xv.at[r]], lv.at[r], sem)
        pltpu.make_async_copy(lse_hbm.at[xv.at[r]], lv.at[r], sem).wait()
    acc = jnp.zeros((16,), jnp.float32)
    for r in range(2):
        for j in range(8):
            acc = acc + lv[r, pl.ds(j * 16, 16)]
    accv[...] = acc
    pltpu.sync_copy(accv, out_hbm.at[wid])


def _loss_kernel(pl_ref, pv_ref, loss_ref):
    tot = jnp.sum(pl_ref[...]) - jnp.sum(pv_ref[...])
    loss_ref[...] = jnp.full((1, 1), tot / (_NW * _TPW), jnp.float32)


def kernel(x, y, table):
    xf = x.reshape(-1).astype(jnp.int32)
    yf = y.reshape(-1).astype(jnp.int32)
    ntok = xf.shape[0]
    mesh = plsc.VectorSubcoreMesh(core_axis_name="c", subcore_axis_name="s")

    x2 = xf.reshape(_NW, _NCHUNK, _CH)
    ycol = jnp.tile(yf.reshape(_NW, _NCHUNK, _CH), (1, 1, 16 // _CH))
    gather = pl.kernel(
        _gather_body,
        out_type=[
            jax.ShapeDtypeStruct((ntok, _V), jnp.float32),
            jax.ShapeDtypeStruct((_NW, 16), jnp.float32),
        ],
        mesh=mesh,
        scratch_types=[
            pltpu.VMEM((_NCHUNK, _CH), jnp.int32),
            pltpu.VMEM((_NCHUNK, 16), jnp.int32),
            pltpu.VMEM((_CH, _V), jnp.float32),
            pltpu.VMEM((_CH, _V), jnp.float32),
            pltpu.VMEM((16,), jnp.float32),
            pltpu.SemaphoreType.DMA,
            pltpu.SemaphoreType.DMA,
            pltpu.SemaphoreType.DMA,
            pltpu.SemaphoreType.DMA,
        ],
    )
    logits, val_parts = gather(x2, ycol, table)

    _RB = 256
    lse = pl.pallas_call(
        _lse_kernel,
        grid=(_V // _RB,),
        in_specs=[pl.BlockSpec((_RB, _V), lambda i: (i, 0))],
        out_specs=pl.BlockSpec((_RB, 1), lambda i: (i, 0)),
        out_shape=jax.ShapeDtypeStruct((_V, 1), jnp.float32),
    )(table)

    x3 = xf.reshape(_NW, 2, 128)
    lse_gather = pl.kernel(
        _lse_gather_body,
        out_type=jax.ShapeDtypeStruct((_NW, 16), jnp.float32),
        mesh=mesh,
        scratch_types=[
            pltpu.VMEM((2, 128), jnp.int32),
            pltpu.VMEM((2, 128), jnp.float32),
            pltpu.VMEM((16,), jnp.float32),
            pltpu.SemaphoreType.DMA,
        ],
    )
    lse_parts = lse_gather(x3, lse.reshape(_V))

    loss = pl.pallas_call(
        _loss_kernel,
        out_shape=jax.ShapeDtypeStruct((1, 1), jnp.float32),
    )(lse_parts, val_parts)

    return (logits, loss[0, 0])
